# baseline (speedup 1.0000x reference)
import functools

import jax
import jax.numpy as jnp
from jax import lax
from jax.experimental import pallas as pl
from jax.experimental.pallas import tpu as pltpu
from jax.experimental.pallas import tpu_sc as plsc

_NUM_EMB = 1024
_COMMIT = 0.25
_EPS = 1e-10

_NC = 2
_NS = 16
_L = 16


def _dist_argmin(x_b, cb):
    xt = jnp.transpose(x_b, (1, 0))
    xsq = jnp.sum(xt * xt, axis=1, keepdims=True)
    cbsq = jnp.sum(cb * cb, axis=1)
    mm = lax.dot_general(xt, cb, (((1,), (1,)), ((), ())),
                         preferred_element_type=jnp.float32)
    dist = (xsq + cbsq[None, :]) - 2.0 * mm

    min_d = jnp.min(dist, axis=1, keepdims=True)
    colsf = lax.broadcasted_iota(jnp.int32, dist.shape, 1).astype(jnp.float32)
    idxf = jnp.min(jnp.where(dist == min_d, colsf, float(_NUM_EMB)), axis=1)
    return min_d, idxf


def _onehot_hist(idxf, n_tok):
    rowsf_cm = lax.broadcasted_iota(
        jnp.int32, (_NUM_EMB, n_tok), 0).astype(jnp.float32)
    onehot_cm = (rowsf_cm == idxf[None, :]).astype(jnp.float32)
    ones_col = jnp.ones((n_tok, 1), jnp.float32)
    hist = jnp.dot(onehot_cm, ones_col,
                   preferred_element_type=jnp.float32)
    return onehot_cm, hist


def _half1_body(x_ref, cb_ref, idx_ref, loss_ref, hist_ref):
    i = pl.program_id(0)

    @pl.when(i == 0)
    def _init():
        loss_ref[...] = jnp.zeros_like(loss_ref)
        hist_ref[...] = jnp.zeros_like(hist_ref)

    x_b = x_ref[0]
    cb = cb_ref[...]
    min_d, idxf = _dist_argmin(x_b, cb)
    idx_ref[0, 0] = idxf.astype(jnp.int32)

    _, hist = _onehot_hist(idxf, x_b.shape[1])
    loss_ref[...] += jnp.sum(min_d, axis=0, keepdims=True)
    hist_ref[...] += hist


def _half2_body(x_ref, cb_ref, lp_ref, hp_ref, out_ref, loss_ref, perp_ref,
                hist_ref):
    i = pl.program_id(0)
    nb = pl.num_programs(0)

    @pl.when(i == 0)
    def _init():
        loss_ref[...] = jnp.zeros_like(loss_ref)
        hist_ref[...] = jnp.zeros_like(hist_ref)
        perp_ref[...] = jnp.zeros_like(perp_ref)

    x_b = x_ref[0]
    cb = cb_ref[...]
    min_d, idxf = _dist_argmin(x_b, cb)
    onehot_cm, hist = _onehot_hist(idxf, x_b.shape[1])
    out_ref[0] = lax.dot_general(cb, onehot_cm, (((0,), (0,)), ((), ())),
                                 preferred_element_type=jnp.float32)

    loss_ref[...] += jnp.sum(min_d, axis=0, keepdims=True)
    hist_ref[...] += hist

    @pl.when(i == nb - 1)
    def _final():
        n_tok = jnp.float32(2 * nb * x_b.shape[1])
        total = n_tok * jnp.float32(x_b.shape[0])
        loss_ref[...] = (1.0 + _COMMIT) * (loss_ref[...] + lp_ref[...]) / total
        p = (hist_ref[...] + hp_ref[...]) / n_tok
        ent = jnp.sum(p * jnp.log(p + _EPS), axis=0, keepdims=True)
        perp_ref[...] = jnp.exp(-ent)


def _tc_half1(x_half, codebook):
    bh, c, n = x_half.shape
    return pl.pallas_call(
        _half1_body,
        grid=(bh,),
        in_specs=[
            pl.BlockSpec((1, c, n), lambda i: (i, 0, 0)),
            pl.BlockSpec((_NUM_EMB, c), lambda i: (0, 0)),
        ],
        out_specs=[
            pl.BlockSpec((1, 1, n), lambda i: (i, 0, 0)),
            pl.BlockSpec((1, 1), lambda i: (0, 0)),
            pl.BlockSpec((_NUM_EMB, 1), lambda i: (0, 0)),
        ],
        out_shape=[
            jax.ShapeDtypeStruct((bh, 1, n), jnp.int32),
            jax.ShapeDtypeStruct((1, 1), jnp.float32),
            jax.ShapeDtypeStruct((_NUM_EMB, 1), jnp.float32),
        ],
    )(x_half, codebook)


def _tc_half2(x_half, codebook, loss_part, hist_part):
    bh, c, n = x_half.shape
    return pl.pallas_call(
        _half2_body,
        grid=(bh,),
        in_specs=[
            pl.BlockSpec((1, c, n), lambda i: (i, 0, 0)),
            pl.BlockSpec((_NUM_EMB, c), lambda i: (0, 0)),
            pl.BlockSpec((1, 1), lambda i: (0, 0)),
            pl.BlockSpec((_NUM_EMB, 1), lambda i: (0, 0)),
        ],
        out_specs=[
            pl.BlockSpec((1, c, n), lambda i: (i, 0, 0)),
            pl.BlockSpec((1, 1), lambda i: (0, 0)),
            pl.BlockSpec((1, 1), lambda i: (0, 0)),
            pl.BlockSpec((_NUM_EMB, 1), lambda i: (0, 0)),
        ],
        out_shape=[
            jax.ShapeDtypeStruct((bh, c, n), jnp.float32),
            jax.ShapeDtypeStruct((1, 1), jnp.float32),
            jax.ShapeDtypeStruct((1, 1), jnp.float32),
            jax.ShapeDtypeStruct((_NUM_EMB, 1), jnp.float32),
        ],
    )(x_half, codebook, loss_part, hist_part)


def _make_sc_gather(bh, c, n):
    n_groups = n // _L
    mesh = plsc.VectorSubcoreMesh(core_axis_name="c", subcore_axis_name="s",
                                  num_cores=_NC, num_subcores=_NS)

    @functools.partial(
        pl.kernel,
        out_type=jax.ShapeDtypeStruct((bh, c * n), jnp.float32),
        mesh=mesh,
        scratch_types=[
            pltpu.VMEM((n,), jnp.int32),
            pltpu.VMEM((c * _NUM_EMB,), jnp.float32),
            pltpu.VMEM((c * n,), jnp.float32),
        ],
        compiler_params=pltpu.CompilerParams(needs_layout_passes=False),
    )
    def sc_gather(cbt_hbm, idx_hbm, out_hbm, idx_v, cbt_v, out_v):
        w = lax.axis_index("s") * _NC + lax.axis_index("c")

        @pl.when(w < bh)
        def _work():
            pltpu.sync_copy(idx_hbm.at[w, 0], idx_v)
            pltpu.sync_copy(cbt_hbm, cbt_v)

            @plsc.parallel_loop(0, n_groups, 1, unroll=2)
            def transpose_group(t0):
                idx16 = idx_v[pl.ds(t0 * _L, _L)]
                for ch in range(c):
                    v = plsc.load_gather(
                        cbt_v.at[pl.ds(ch * _NUM_EMB, _NUM_EMB)], [idx16])
                    out_v[pl.ds(ch * n + t0 * _L, _L)] = v
            pltpu.sync_copy(out_v, out_hbm.at[w])

    return sc_gather


def kernel(inputs, codebook):
    b, c, n = inputs.shape
    bh = b // 2
    cbt_flat = jnp.transpose(codebook, (1, 0)).reshape(-1)

    idx1, loss1, hist1 = _tc_half1(inputs[:bh], codebook)
    out1 = _make_sc_gather(bh, c, n)(cbt_flat, idx1).reshape(bh, c, n)
    out2, loss, perp, _hist = _tc_half2(inputs[bh:], codebook, loss1, hist1)
    out_q = jnp.concatenate([out1, out2], axis=0)
    return (loss[0, 0], out_q, perp[0, 0])


# speedup vs baseline: 1.0679x; 1.0679x over previous
import functools

import jax
import jax.numpy as jnp
from jax import lax
from jax.experimental import pallas as pl
from jax.experimental.pallas import tpu as pltpu
from jax.experimental.pallas import tpu_sc as plsc

_NUM_EMB = 1024
_COMMIT = 0.25
_EPS = 1e-10

_NC = 2
_NS = 16
_L = 16


def _dist_argmin(x_b, cb):
    xt = jnp.transpose(x_b, (1, 0))
    xsq = jnp.sum(xt * xt, axis=1, keepdims=True)
    cbsq = jnp.sum(cb * cb, axis=1)
    mm = lax.dot_general(xt, cb, (((1,), (1,)), ((), ())),
                         preferred_element_type=jnp.float32)
    dist = (xsq + cbsq[None, :]) - 2.0 * mm

    min_d = jnp.min(dist, axis=1, keepdims=True)
    colsf = lax.broadcasted_iota(jnp.int32, dist.shape, 1).astype(jnp.float32)
    idxf = jnp.min(jnp.where(dist == min_d, colsf, float(_NUM_EMB)), axis=1)
    return min_d, idxf


def _onehot_hist(idxf, n_tok):
    rowsf_cm = lax.broadcasted_iota(
        jnp.int32, (_NUM_EMB, n_tok), 0).astype(jnp.float32)
    onehot_cm = (rowsf_cm == idxf[None, :]).astype(jnp.float32)
    ones_col = jnp.ones((n_tok, 1), jnp.float32)
    hist = jnp.dot(onehot_cm, ones_col,
                   preferred_element_type=jnp.float32)
    return onehot_cm, hist


def _half1_body(x_ref, cb_ref, idx_ref, loss_ref, hist_ref):
    i = pl.program_id(0)

    @pl.when(i == 0)
    def _init():
        loss_ref[...] = jnp.zeros_like(loss_ref)
        hist_ref[...] = jnp.zeros_like(hist_ref)

    x_b = x_ref[0]
    cb = cb_ref[...]
    min_d, idxf = _dist_argmin(x_b, cb)
    idx_ref[0, 0] = idxf.astype(jnp.int32)

    _, hist = _onehot_hist(idxf, x_b.shape[1])
    loss_ref[...] += jnp.sum(min_d, axis=0, keepdims=True)
    hist_ref[...] += hist


def _half2_body(x_ref, cb_ref, lp_ref, hp_ref, out_ref, loss_ref, perp_ref,
                hist_ref):
    i = pl.program_id(0)
    nb = pl.num_programs(0)

    @pl.when(i == 0)
    def _init():
        loss_ref[...] = jnp.zeros_like(loss_ref)
        hist_ref[...] = jnp.zeros_like(hist_ref)
        perp_ref[...] = jnp.zeros_like(perp_ref)

    x_b = x_ref[0]
    cb = cb_ref[...]
    min_d, idxf = _dist_argmin(x_b, cb)
    onehot_cm, hist = _onehot_hist(idxf, x_b.shape[1])
    out_ref[0] = lax.dot_general(cb, onehot_cm, (((0,), (0,)), ((), ())),
                                 preferred_element_type=jnp.float32)

    loss_ref[...] += jnp.sum(min_d, axis=0, keepdims=True)
    hist_ref[...] += hist

    @pl.when(i == nb - 1)
    def _final():
        n_tok = jnp.float32(2 * nb * x_b.shape[1])
        total = n_tok * jnp.float32(x_b.shape[0])
        loss_ref[...] = (1.0 + _COMMIT) * (loss_ref[...] + lp_ref[...]) / total
        p = (hist_ref[...] + hp_ref[...]) / n_tok
        ent = jnp.sum(p * jnp.log(p + _EPS), axis=0, keepdims=True)
        perp_ref[...] = jnp.exp(-ent)


def _tc_half1(x_full, codebook, bh):
    _, c, n = x_full.shape
    return pl.pallas_call(
        _half1_body,
        grid=(bh,),
        in_specs=[
            pl.BlockSpec((1, c, n), lambda i: (i, 0, 0)),
            pl.BlockSpec((_NUM_EMB, c), lambda i: (0, 0)),
        ],
        out_specs=[
            pl.BlockSpec((1, 1, n), lambda i: (i, 0, 0)),
            pl.BlockSpec((1, 1), lambda i: (0, 0)),
            pl.BlockSpec((_NUM_EMB, 1), lambda i: (0, 0)),
        ],
        out_shape=[
            jax.ShapeDtypeStruct((bh, 1, n), jnp.int32),
            jax.ShapeDtypeStruct((1, 1), jnp.float32),
            jax.ShapeDtypeStruct((_NUM_EMB, 1), jnp.float32),
        ],
    )(x_full, codebook)


def _tc_half2(x_full, codebook, loss_part, hist_part, bh):
    _, c, n = x_full.shape
    return pl.pallas_call(
        _half2_body,
        grid=(bh,),
        in_specs=[
            pl.BlockSpec((1, c, n), lambda i: (i + bh, 0, 0)),
            pl.BlockSpec((_NUM_EMB, c), lambda i: (0, 0)),
            pl.BlockSpec((1, 1), lambda i: (0, 0)),
            pl.BlockSpec((_NUM_EMB, 1), lambda i: (0, 0)),
        ],
        out_specs=[
            pl.BlockSpec((1, c, n), lambda i: (i, 0, 0)),
            pl.BlockSpec((1, 1), lambda i: (0, 0)),
            pl.BlockSpec((1, 1), lambda i: (0, 0)),
            pl.BlockSpec((_NUM_EMB, 1), lambda i: (0, 0)),
        ],
        out_shape=[
            jax.ShapeDtypeStruct((bh, c, n), jnp.float32),
            jax.ShapeDtypeStruct((1, 1), jnp.float32),
            jax.ShapeDtypeStruct((1, 1), jnp.float32),
            jax.ShapeDtypeStruct((_NUM_EMB, 1), jnp.float32),
        ],
    )(x_full, codebook, loss_part, hist_part)


def _make_sc_gather(bh, c, n):
    n_groups = n // _L
    mesh = plsc.VectorSubcoreMesh(core_axis_name="c", subcore_axis_name="s",
                                  num_cores=_NC, num_subcores=_NS)

    @functools.partial(
        pl.kernel,
        out_type=jax.ShapeDtypeStruct((bh, c, n), jnp.float32),
        mesh=mesh,
        scratch_types=[
            pltpu.VMEM((n,), jnp.int32),
            pltpu.VMEM((c * _NUM_EMB,), jnp.float32),
            pltpu.VMEM((c, n), jnp.float32),
        ],
        compiler_params=pltpu.CompilerParams(needs_layout_passes=False),
    )
    def sc_gather(cbt_hbm, idx_hbm, out_hbm, idx_v, cbt_v, out_v):
        w = lax.axis_index("s") * _NC + lax.axis_index("c")

        @pl.when(w < bh)
        def _work():
            pltpu.sync_copy(idx_hbm.at[w, 0], idx_v)
            pltpu.sync_copy(cbt_hbm, cbt_v)

            def per_group(t0, carry):
                idx16 = idx_v[pl.ds(t0 * _L, _L)]

                @plsc.parallel_loop(0, c, 1, unroll=4)
                def per_channel(ch):
                    v = plsc.load_gather(
                        cbt_v.at[pl.ds(ch * _NUM_EMB, _NUM_EMB)], [idx16])
                    out_v[ch, pl.ds(t0 * _L, _L)] = v
                return carry

            lax.fori_loop(0, n_groups, per_group, 0)
            pltpu.sync_copy(out_v, out_hbm.at[w])

    return sc_gather


def kernel(inputs, codebook):
    b, c, n = inputs.shape
    bh = b // 2
    cbt_flat = jnp.transpose(codebook, (1, 0)).reshape(-1)

    idx1, loss1, hist1 = _tc_half1(inputs, codebook, bh)
    out1 = _make_sc_gather(bh, c, n)(cbt_flat, idx1)
    out2, loss, perp, _hist = _tc_half2(inputs, codebook, loss1, hist1, bh)
    out_q = jnp.concatenate([out1, out2], axis=0)
    return (loss[0, 0], out_q, perp[0, 0])


# speedup vs baseline: 1.0897x; 1.0204x over previous
import functools

import jax
import jax.numpy as jnp
from jax import lax
from jax.experimental import pallas as pl
from jax.experimental.pallas import tpu as pltpu
from jax.experimental.pallas import tpu_sc as plsc

_NUM_EMB = 1024
_COMMIT = 0.25
_EPS = 1e-10

_NC = 2
_NS = 16
_L = 16


def _dist_argmin(x_b, cb):
    xt = jnp.transpose(x_b, (1, 0))
    xsq = jnp.sum(xt * xt, axis=1, keepdims=True)
    cbsq = jnp.sum(cb * cb, axis=1)
    mm = lax.dot_general(xt, cb, (((1,), (1,)), ((), ())),
                         preferred_element_type=jnp.float32)
    dist = (xsq + cbsq[None, :]) - 2.0 * mm

    min_d = jnp.min(dist, axis=1, keepdims=True)
    colsf = lax.broadcasted_iota(jnp.int32, dist.shape, 1).astype(jnp.float32)
    idxf = jnp.min(jnp.where(dist == min_d, colsf, float(_NUM_EMB)), axis=1)
    return min_d, idxf


def _onehot_hist(idxf, n_tok):
    rowsf_cm = lax.broadcasted_iota(
        jnp.int32, (_NUM_EMB, n_tok), 0).astype(jnp.float32)
    onehot_cm = (rowsf_cm == idxf[None, :]).astype(jnp.float32)
    ones_col = jnp.ones((n_tok, 1), jnp.float32)
    hist = jnp.dot(onehot_cm, ones_col,
                   preferred_element_type=jnp.float32)
    return onehot_cm, hist


def _half1_body(x_ref, cb_ref, idx_ref, loss_ref, hist_ref):
    i = pl.program_id(0)

    @pl.when(i == 0)
    def _init():
        loss_ref[...] = jnp.zeros_like(loss_ref)
        hist_ref[...] = jnp.zeros_like(hist_ref)

    x_b = x_ref[0]
    cb = cb_ref[...]
    min_d, idxf = _dist_argmin(x_b, cb)
    idx_ref[0, 0] = idxf.astype(jnp.int32)

    _, hist = _onehot_hist(idxf, x_b.shape[1])
    loss_ref[...] += jnp.sum(min_d, axis=0, keepdims=True)
    hist_ref[...] += hist


def _half2_body(x_ref, cb_ref, lp_ref, hp_ref, out_ref, loss_ref, perp_ref,
                hist_ref):
    i = pl.program_id(0)
    nb = pl.num_programs(0)

    @pl.when(i == 0)
    def _init():
        loss_ref[...] = jnp.zeros_like(loss_ref)
        hist_ref[...] = jnp.zeros_like(hist_ref)
        perp_ref[...] = jnp.zeros_like(perp_ref)

    x_b = x_ref[0]
    cb = cb_ref[...]
    min_d, idxf = _dist_argmin(x_b, cb)
    onehot_cm, hist = _onehot_hist(idxf, x_b.shape[1])
    out_ref[0] = lax.dot_general(cb, onehot_cm, (((0,), (0,)), ((), ())),
                                 preferred_element_type=jnp.float32)

    loss_ref[...] += jnp.sum(min_d, axis=0, keepdims=True)
    hist_ref[...] += hist

    @pl.when(i == nb - 1)
    def _final():
        n_tok = jnp.float32(2 * nb * x_b.shape[1])
        total = n_tok * jnp.float32(x_b.shape[0])
        loss_ref[...] = (1.0 + _COMMIT) * (loss_ref[...] + lp_ref[...]) / total
        p = (hist_ref[...] + hp_ref[...]) / n_tok
        ent = jnp.sum(p * jnp.log(p + _EPS), axis=0, keepdims=True)
        perp_ref[...] = jnp.exp(-ent)


def _tc_half1(x_full, codebook, bh):
    _, c, n = x_full.shape
    return pl.pallas_call(
        _half1_body,
        grid=(bh,),
        in_specs=[
            pl.BlockSpec((1, c, n), lambda i: (i, 0, 0)),
            pl.BlockSpec((_NUM_EMB, c), lambda i: (0, 0)),
        ],
        out_specs=[
            pl.BlockSpec((1, 1, n), lambda i: (i, 0, 0)),
            pl.BlockSpec((1, 1), lambda i: (0, 0)),
            pl.BlockSpec((_NUM_EMB, 1), lambda i: (0, 0)),
        ],
        out_shape=[
            jax.ShapeDtypeStruct((bh, 1, n), jnp.int32),
            jax.ShapeDtypeStruct((1, 1), jnp.float32),
            jax.ShapeDtypeStruct((_NUM_EMB, 1), jnp.float32),
        ],
    )(x_full, codebook)


def _tc_half2(x_full, codebook, loss_part, hist_part, bh):
    _, c, n = x_full.shape
    return pl.pallas_call(
        _half2_body,
        grid=(bh,),
        in_specs=[
            pl.BlockSpec((1, c, n), lambda i: (i + bh, 0, 0)),
            pl.BlockSpec((_NUM_EMB, c), lambda i: (0, 0)),
            pl.BlockSpec((1, 1), lambda i: (0, 0)),
            pl.BlockSpec((_NUM_EMB, 1), lambda i: (0, 0)),
        ],
        out_specs=[
            pl.BlockSpec((1, c, n), lambda i: (i, 0, 0)),
            pl.BlockSpec((1, 1), lambda i: (0, 0)),
            pl.BlockSpec((1, 1), lambda i: (0, 0)),
            pl.BlockSpec((_NUM_EMB, 1), lambda i: (0, 0)),
        ],
        out_shape=[
            jax.ShapeDtypeStruct((bh, c, n), jnp.float32),
            jax.ShapeDtypeStruct((1, 1), jnp.float32),
            jax.ShapeDtypeStruct((1, 1), jnp.float32),
            jax.ShapeDtypeStruct((_NUM_EMB, 1), jnp.float32),
        ],
    )(x_full, codebook, loss_part, hist_part)


def _make_sc_gather(bh, c, n):
    n_groups = n // _L
    mesh = plsc.VectorSubcoreMesh(core_axis_name="c", subcore_axis_name="s",
                                  num_cores=_NC, num_subcores=_NS)

    @functools.partial(
        pl.kernel,
        out_type=jax.ShapeDtypeStruct((2 * bh, c, n), jnp.float32),
        mesh=mesh,
        scratch_types=[
            pltpu.VMEM((n,), jnp.int32),
            pltpu.VMEM((c * _NUM_EMB,), jnp.float32),
            pltpu.VMEM((c, n), jnp.float32),
        ],
        compiler_params=pltpu.CompilerParams(needs_layout_passes=False),
    )
    def sc_gather(cbt_hbm, idx_hbm, out_hbm, idx_v, cbt_v, out_v):
        w = lax.axis_index("s") * _NC + lax.axis_index("c")

        @pl.when(w < bh)
        def _work():
            pltpu.sync_copy(idx_hbm.at[w, 0], idx_v)
            pltpu.sync_copy(cbt_hbm, cbt_v)

            def per_group(t0, carry):
                idx16 = idx_v[pl.ds(t0 * _L, _L)]

                @plsc.parallel_loop(0, c, 1, unroll=4)
                def per_channel(ch):
                    v = plsc.load_gather(
                        cbt_v.at[pl.ds(ch * _NUM_EMB, _NUM_EMB)], [idx16])
                    out_v[ch, pl.ds(t0 * _L, _L)] = v
                return carry

            lax.fori_loop(0, n_groups, per_group, 0)
            pltpu.sync_copy(out_v, out_hbm.at[w])

    return sc_gather


def kernel(inputs, codebook):
    b, c, n = inputs.shape
    bh = b // 2
    cbt_flat = jnp.transpose(codebook, (1, 0)).reshape(-1)

    idx1, loss1, hist1 = _tc_half1(inputs, codebook, bh)
    out_full = _make_sc_gather(bh, c, n)(cbt_flat, idx1)
    out2, loss, perp, _hist = _tc_half2(inputs, codebook, loss1, hist1, bh)
    out_q = lax.dynamic_update_slice(out_full, out2, (bh, 0, 0))
    return (loss[0, 0], out_q, perp[0, 0])


# speedup vs baseline: 1.1417x; 1.0478x over previous
import functools

import jax
import jax.numpy as jnp
from jax import lax
from jax.experimental import pallas as pl
from jax.experimental.pallas import tpu as pltpu
from jax.experimental.pallas import tpu_sc as plsc

_NUM_EMB = 1024
_COMMIT = 0.25
_EPS = 1e-10

_NC = 2
_NS = 16
_L = 16


def _dist_argmin(x_b, cb):
    xt = jnp.transpose(x_b, (1, 0))
    xsq = jnp.sum(xt * xt, axis=1, keepdims=True)
    cbsq = jnp.sum(cb * cb, axis=1)
    mm = lax.dot_general(xt, cb, (((1,), (1,)), ((), ())),
                         preferred_element_type=jnp.float32)
    dist = (xsq + cbsq[None, :]) - 2.0 * mm

    min_d = jnp.min(dist, axis=1, keepdims=True)
    colsf = lax.broadcasted_iota(jnp.int32, dist.shape, 1).astype(jnp.float32)
    idxf = jnp.min(jnp.where(dist == min_d, colsf, float(_NUM_EMB)), axis=1)
    return min_d, idxf


def _onehot_hist(idxf, n_tok):
    rowsf_cm = lax.broadcasted_iota(
        jnp.int32, (_NUM_EMB, n_tok), 0).astype(jnp.float32)
    onehot_cm = (rowsf_cm == idxf[None, :]).astype(jnp.float32)
    ones_col = jnp.ones((n_tok, 1), jnp.float32)
    hist = jnp.dot(onehot_cm, ones_col,
                   preferred_element_type=jnp.float32)
    return onehot_cm, hist


def _half1_body(x_ref, cb_ref, idx_ref, loss_ref, hist_ref):
    i = pl.program_id(0)

    @pl.when(i == 0)
    def _init():
        loss_ref[...] = jnp.zeros_like(loss_ref)
        hist_ref[...] = jnp.zeros_like(hist_ref)

    cb = cb_ref[...]
    for j in range(x_ref.shape[0]):
        x_b = x_ref[j]
        min_d, idxf = _dist_argmin(x_b, cb)
        idx_ref[j, 0] = idxf.astype(jnp.int32)

        _, hist = _onehot_hist(idxf, x_b.shape[1])
        loss_ref[...] += jnp.sum(min_d, axis=0, keepdims=True)
        hist_ref[...] += hist


def _half2_body(x_ref, cb_ref, lp_ref, hp_ref, out_ref, loss_ref, perp_ref,
                hist_ref):
    i = pl.program_id(0)
    nb = pl.num_programs(0)

    @pl.when(i == 0)
    def _init():
        loss_ref[...] = jnp.zeros_like(loss_ref)
        hist_ref[...] = jnp.zeros_like(hist_ref)
        perp_ref[...] = jnp.zeros_like(perp_ref)

    cb = cb_ref[...]
    for j in range(x_ref.shape[0]):
        x_b = x_ref[j]
        min_d, idxf = _dist_argmin(x_b, cb)
        onehot_cm, hist = _onehot_hist(idxf, x_b.shape[1])
        out_ref[j] = lax.dot_general(cb, onehot_cm, (((0,), (0,)), ((), ())),
                                     preferred_element_type=jnp.float32)

        loss_ref[...] += jnp.sum(min_d, axis=0, keepdims=True)
        hist_ref[...] += hist

    @pl.when(i == nb - 1)
    def _final():
        bps = x_ref.shape[0]
        n_tok = jnp.float32(2 * nb * bps * x_ref.shape[2])
        total = n_tok * jnp.float32(x_ref.shape[1])
        loss_ref[...] = (1.0 + _COMMIT) * (loss_ref[...] + lp_ref[...]) / total
        p = (hist_ref[...] + hp_ref[...]) / n_tok
        ent = jnp.sum(p * jnp.log(p + _EPS), axis=0, keepdims=True)
        perp_ref[...] = jnp.exp(-ent)


def _tc_half1(x_full, codebook, bh, bps=2):
    _, c, n = x_full.shape
    return pl.pallas_call(
        _half1_body,
        grid=(bh // bps,),
        in_specs=[
            pl.BlockSpec((bps, c, n), lambda i: (i, 0, 0)),
            pl.BlockSpec((_NUM_EMB, c), lambda i: (0, 0)),
        ],
        out_specs=[
            pl.BlockSpec((bps, 1, n), lambda i: (i, 0, 0)),
            pl.BlockSpec((1, 1), lambda i: (0, 0)),
            pl.BlockSpec((_NUM_EMB, 1), lambda i: (0, 0)),
        ],
        out_shape=[
            jax.ShapeDtypeStruct((bh, 1, n), jnp.int32),
            jax.ShapeDtypeStruct((1, 1), jnp.float32),
            jax.ShapeDtypeStruct((_NUM_EMB, 1), jnp.float32),
        ],
    )(x_full, codebook)


def _tc_half2(x_full, codebook, loss_part, hist_part, bh, bps=2):
    _, c, n = x_full.shape
    nsteps = bh // bps
    return pl.pallas_call(
        _half2_body,
        grid=(nsteps,),
        in_specs=[
            pl.BlockSpec((bps, c, n), lambda i: (i + nsteps, 0, 0)),
            pl.BlockSpec((_NUM_EMB, c), lambda i: (0, 0)),
            pl.BlockSpec((1, 1), lambda i: (0, 0)),
            pl.BlockSpec((_NUM_EMB, 1), lambda i: (0, 0)),
        ],
        out_specs=[
            pl.BlockSpec((bps, c, n), lambda i: (i, 0, 0)),
            pl.BlockSpec((1, 1), lambda i: (0, 0)),
            pl.BlockSpec((1, 1), lambda i: (0, 0)),
            pl.BlockSpec((_NUM_EMB, 1), lambda i: (0, 0)),
        ],
        out_shape=[
            jax.ShapeDtypeStruct((bh, c, n), jnp.float32),
            jax.ShapeDtypeStruct((1, 1), jnp.float32),
            jax.ShapeDtypeStruct((1, 1), jnp.float32),
            jax.ShapeDtypeStruct((_NUM_EMB, 1), jnp.float32),
        ],
    )(x_full, codebook, loss_part, hist_part)


def _make_sc_gather(bh, c, n):
    n_groups = n // _L
    mesh = plsc.VectorSubcoreMesh(core_axis_name="c", subcore_axis_name="s",
                                  num_cores=_NC, num_subcores=_NS)

    @functools.partial(
        pl.kernel,
        out_type=jax.ShapeDtypeStruct((2 * bh, c, n), jnp.float32),
        mesh=mesh,
        scratch_types=[
            pltpu.VMEM((n,), jnp.int32),
            pltpu.VMEM((c * _NUM_EMB,), jnp.float32),
            pltpu.VMEM((c, n), jnp.float32),
        ],
        compiler_params=pltpu.CompilerParams(needs_layout_passes=False),
    )
    def sc_gather(cbt_hbm, idx_hbm, out_hbm, idx_v, cbt_v, out_v):
        w = lax.axis_index("s") * _NC + lax.axis_index("c")

        @pl.when(w < bh)
        def _work():
            pltpu.sync_copy(idx_hbm.at[w, 0], idx_v)
            pltpu.sync_copy(cbt_hbm, cbt_v)

            def per_group(t0, carry):
                idx16 = idx_v[pl.ds(t0 * _L, _L)]

                @plsc.parallel_loop(0, c, 1, unroll=4)
                def per_channel(ch):
                    v = plsc.load_gather(
                        cbt_v.at[pl.ds(ch * _NUM_EMB, _NUM_EMB)], [idx16])
                    out_v[ch, pl.ds(t0 * _L, _L)] = v
                return carry

            lax.fori_loop(0, n_groups, per_group, 0)
            pltpu.sync_copy(out_v, out_hbm.at[w])

    return sc_gather


def kernel(inputs, codebook):
    b, c, n = inputs.shape
    bh = b // 2
    cbt_flat = jnp.transpose(codebook, (1, 0)).reshape(-1)

    idx1, loss1, hist1 = _tc_half1(inputs, codebook, bh)
    out_full = _make_sc_gather(bh, c, n)(cbt_flat, idx1)
    out2, loss, perp, _hist = _tc_half2(inputs, codebook, loss1, hist1, bh)
    out_q = lax.dynamic_update_slice(out_full, out2, (bh, 0, 0))
    return (loss[0, 0], out_q, perp[0, 0])


# speedup vs baseline: 1.1654x; 1.0207x over previous
import functools

import jax
import jax.numpy as jnp
from jax import lax
from jax.experimental import pallas as pl
from jax.experimental.pallas import tpu as pltpu
from jax.experimental.pallas import tpu_sc as plsc

_NUM_EMB = 1024
_COMMIT = 0.25
_EPS = 1e-10

_NC = 2
_NS = 16
_L = 16


def _dist_argmin(x_b, cb):
    xt = jnp.transpose(x_b, (1, 0))
    xsq = jnp.sum(xt * xt, axis=1, keepdims=True)
    cbsq = jnp.sum(cb * cb, axis=1)
    mm = lax.dot_general(xt, cb, (((1,), (1,)), ((), ())),
                         preferred_element_type=jnp.float32)
    dist = (xsq + cbsq[None, :]) - 2.0 * mm

    min_d = jnp.min(dist, axis=1, keepdims=True)
    colsf = lax.broadcasted_iota(jnp.int32, dist.shape, 1).astype(jnp.float32)
    idxf = jnp.min(jnp.where(dist == min_d, colsf, float(_NUM_EMB)), axis=1)
    return min_d, idxf


def _onehot_hist(idxf, n_tok):
    rowsf_cm = lax.broadcasted_iota(
        jnp.int32, (_NUM_EMB, n_tok), 0).astype(jnp.float32)
    onehot_cm = (rowsf_cm == idxf[None, :]).astype(jnp.float32)
    ones_col = jnp.ones((n_tok, 1), jnp.float32)
    hist = jnp.dot(onehot_cm, ones_col,
                   preferred_element_type=jnp.float32)
    return onehot_cm, hist


def _half1_body(x_ref, cb_ref, idx_ref, loss_ref, hist_ref):
    i = pl.program_id(0)

    @pl.when(i == 0)
    def _init():
        loss_ref[...] = jnp.zeros_like(loss_ref)
        hist_ref[...] = jnp.zeros_like(hist_ref)

    cb = cb_ref[...]
    for j in range(x_ref.shape[0]):
        x_b = x_ref[j]
        min_d, idxf = _dist_argmin(x_b, cb)
        idx_ref[j, 0] = idxf.astype(jnp.int32)

        _, hist = _onehot_hist(idxf, x_b.shape[1])
        loss_ref[...] += jnp.sum(min_d, axis=0, keepdims=True)
        hist_ref[...] += hist


def _half2_body(x_ref, cb_ref, lp_ref, hp_ref, out_ref, loss_ref, perp_ref,
                hist_ref):
    i = pl.program_id(0)
    nb = pl.num_programs(0)

    @pl.when(i == 0)
    def _init():
        loss_ref[...] = jnp.zeros_like(loss_ref)
        hist_ref[...] = jnp.zeros_like(hist_ref)
        perp_ref[...] = jnp.zeros_like(perp_ref)

    cb = cb_ref[...]
    for j in range(x_ref.shape[0]):
        x_b = x_ref[j]
        min_d, idxf = _dist_argmin(x_b, cb)
        onehot_cm, hist = _onehot_hist(idxf, x_b.shape[1])
        out_ref[j] = lax.dot_general(cb, onehot_cm, (((0,), (0,)), ((), ())),
                                     preferred_element_type=jnp.float32)

        loss_ref[...] += jnp.sum(min_d, axis=0, keepdims=True)
        hist_ref[...] += hist

    @pl.when(i == nb - 1)
    def _final():
        bps = x_ref.shape[0]
        n_tok = jnp.float32(2 * nb * bps * x_ref.shape[2])
        total = n_tok * jnp.float32(x_ref.shape[1])
        loss_ref[...] = (1.0 + _COMMIT) * (loss_ref[...] + lp_ref[...]) / total
        p = (hist_ref[...] + hp_ref[...]) / n_tok
        ent = jnp.sum(p * jnp.log(p + _EPS), axis=0, keepdims=True)
        perp_ref[...] = jnp.exp(-ent)


def _tc_half1(x_full, codebook, bh, bps=4):
    _, c, n = x_full.shape
    return pl.pallas_call(
        _half1_body,
        grid=(bh // bps,),
        in_specs=[
            pl.BlockSpec((bps, c, n), lambda i: (i, 0, 0)),
            pl.BlockSpec((_NUM_EMB, c), lambda i: (0, 0)),
        ],
        out_specs=[
            pl.BlockSpec((bps, 1, n), lambda i: (i, 0, 0)),
            pl.BlockSpec((1, 1), lambda i: (0, 0)),
            pl.BlockSpec((_NUM_EMB, 1), lambda i: (0, 0)),
        ],
        out_shape=[
            jax.ShapeDtypeStruct((bh, 1, n), jnp.int32),
            jax.ShapeDtypeStruct((1, 1), jnp.float32),
            jax.ShapeDtypeStruct((_NUM_EMB, 1), jnp.float32),
        ],
    )(x_full, codebook)


def _tc_half2(x_full, codebook, loss_part, hist_part, bh, bps=4):
    _, c, n = x_full.shape
    nsteps = bh // bps
    return pl.pallas_call(
        _half2_body,
        grid=(nsteps,),
        in_specs=[
            pl.BlockSpec((bps, c, n), lambda i: (i + nsteps, 0, 0)),
            pl.BlockSpec((_NUM_EMB, c), lambda i: (0, 0)),
            pl.BlockSpec((1, 1), lambda i: (0, 0)),
            pl.BlockSpec((_NUM_EMB, 1), lambda i: (0, 0)),
        ],
        out_specs=[
            pl.BlockSpec((bps, c, n), lambda i: (i, 0, 0)),
            pl.BlockSpec((1, 1), lambda i: (0, 0)),
            pl.BlockSpec((1, 1), lambda i: (0, 0)),
            pl.BlockSpec((_NUM_EMB, 1), lambda i: (0, 0)),
        ],
        out_shape=[
            jax.ShapeDtypeStruct((bh, c, n), jnp.float32),
            jax.ShapeDtypeStruct((1, 1), jnp.float32),
            jax.ShapeDtypeStruct((1, 1), jnp.float32),
            jax.ShapeDtypeStruct((_NUM_EMB, 1), jnp.float32),
        ],
    )(x_full, codebook, loss_part, hist_part)


def _make_sc_gather(bh, c, n):
    n_groups = n // _L
    mesh = plsc.VectorSubcoreMesh(core_axis_name="c", subcore_axis_name="s",
                                  num_cores=_NC, num_subcores=_NS)

    @functools.partial(
        pl.kernel,
        out_type=jax.ShapeDtypeStruct((2 * bh, c, n), jnp.float32),
        mesh=mesh,
        scratch_types=[
            pltpu.VMEM((n,), jnp.int32),
            pltpu.VMEM((c * _NUM_EMB,), jnp.float32),
            pltpu.VMEM((c, n), jnp.float32),
        ],
        compiler_params=pltpu.CompilerParams(needs_layout_passes=False),
    )
    def sc_gather(cbt_hbm, idx_hbm, out_hbm, idx_v, cbt_v, out_v):
        w = lax.axis_index("s") * _NC + lax.axis_index("c")

        @pl.when(w < bh)
        def _work():
            pltpu.sync_copy(idx_hbm.at[w, 0], idx_v)
            pltpu.sync_copy(cbt_hbm, cbt_v)

            def per_group(t0, carry):
                idx16 = idx_v[pl.ds(t0 * _L, _L)]

                @plsc.parallel_loop(0, c, 1, unroll=4)
                def per_channel(ch):
                    v = plsc.load_gather(
                        cbt_v.at[pl.ds(ch * _NUM_EMB, _NUM_EMB)], [idx16])
                    out_v[ch, pl.ds(t0 * _L, _L)] = v
                return carry

            lax.fori_loop(0, n_groups, per_group, 0)
            pltpu.sync_copy(out_v, out_hbm.at[w])

    return sc_gather


def kernel(inputs, codebook):
    b, c, n = inputs.shape
    bh = b // 2
    cbt_flat = jnp.transpose(codebook, (1, 0)).reshape(-1)

    idx1, loss1, hist1 = _tc_half1(inputs, codebook, bh)
    out_full = _make_sc_gather(bh, c, n)(cbt_flat, idx1)
    out2, loss, perp, _hist = _tc_half2(inputs, codebook, loss1, hist1, bh)
    out_q = lax.dynamic_update_slice(out_full, out2, (bh, 0, 0))
    return (loss[0, 0], out_q, perp[0, 0])
